# baseline (speedup 1.0000x reference)
import functools

import jax
import jax.numpy as jnp
from jax import lax
from jax.experimental import pallas as pl
from jax.experimental.pallas import tpu as pltpu
from jax.experimental.pallas import tpu_sc as plsc

_B, _N, _MAP, _DY, _DX = 4, 512, 4, 8, 6
_D = 256
_TE = 6
_TD = _TE * _DX
_K = _N * _MAP * _DY
_KT = 2048
_NBLK = _K // _KT
_NC, _NS = 2, 16
_ROWS = 64
_NCD = _D // 16
_NI = 32
_RING = 4


def _tc_t_body(x_ref, t2vw_ref, t2vb_ref, vtw_ref, vtb_ref, given_ref, t_ref):
    x = x_ref[0]
    xn = jnp.where(jnp.isnan(x), 0.0, x)
    xrep = jnp.repeat(xn, _TE, axis=1)
    xa = xrep * t2vw_ref[...] + t2vb_ref[...]
    col = lax.broadcasted_iota(jnp.int32, (_N, _TD), 1)
    tv = jnp.where(col % _TE == 0, xa, jnp.sin(xa))
    tt = jnp.dot(tv, vtw_ref[:_TD, :], preferred_element_type=jnp.float32)
    t_ref[0] = tt + vtb_ref[...] + given_ref[1:2, :]


def _sc_body(t_hbm, local_hbm, yperm_hbm, ygperm_hbm, space_hbm, wrow_hbm,
             given_hbm, val_out, space_out, var_out,
             tbuf0, tbuf1, lbuf0, lbuf1, ybuf0, ybuf1, ygbuf0, ygbuf1,
             outbuf, rowbuf, varbuf, gbuf, idxl0, idxl1,
             sem_pre, sem_val, sem_space):
    wid = lax.axis_index("s") * _NC + lax.axis_index("c")
    b0 = wid // 16
    p0 = lax.rem(wid, 16)
    b1 = (wid + 32) // 16
    p1 = lax.rem(wid + 32, 16)

    bs = wid // _NBLK
    seg = lax.rem(wid, _NBLK)
    pltpu.sync_copy(space_hbm.at[pl.ds(seg, 1)], rowbuf.at[pl.ds(0, 1)])
    svec = [rowbuf[0, pl.ds(d * 16, 16)] for d in range(_NCD)]
    for r in range(1, _ROWS):
        for d in range(_NCD):
            rowbuf[r, pl.ds(d * 16, 16)] = svec[d]
    vv = jnp.full((16,), seg, jnp.int32)
    for q in range(_KT // 16):
        varbuf[pl.ds(q * 16, 16)] = vv
    sbase = seg * _KT
    for i in range(_KT // _ROWS):
        pltpu.async_copy(rowbuf, space_out.at[bs, pl.ds(sbase + i * _ROWS,
                                                        _ROWS)], sem_space)
    pltpu.async_copy(varbuf, var_out.at[bs, pl.ds(sbase, _KT)], sem_space)

    cps = [pltpu.async_copy(wrow_hbm, gbuf.at[pl.ds(0, 1)], sem_pre),
           pltpu.async_copy(given_hbm, gbuf.at[pl.ds(1, 2)], sem_pre)]
    iv = jnp.arange(16, dtype=jnp.int32) * 16
    for g, (b, p, il, tb, lb, yb, ygb) in enumerate(
            ((b0, p0, idxl0, tbuf0, lbuf0, ybuf0, ygbuf0),
             (b1, p1, idxl1, tbuf1, lbuf1, ybuf1, ygbuf1))):
        il[pl.ds(0, 16)] = p + iv
        il[pl.ds(16, 16)] = p + 256 + iv
        cps.append(pltpu.async_copy(t_hbm.at[b, pl.ds(32 * p, 32)], tb,
                                    sem_pre))
        cps.append(pltpu.async_copy(local_hbm.at[il], lb, sem_pre))
        cps.append(pltpu.async_copy(yperm_hbm.at[b, p], yb, sem_pre))
        cps.append(pltpu.async_copy(ygperm_hbm.at[b, p], ygb, sem_pre))
    for cp in cps:
        cp.wait()
    wy = [gbuf[0, pl.ds(d * 16, 16)] for d in range(_NCD)]
    gdiff = [gbuf[1, pl.ds(d * 16, 16)] - gbuf[2, pl.ds(d * 16, 16)]
             for d in range(_NCD)]

    for g, (b, p, tb, lb, yb, ygb) in enumerate(
            ((b0, p0, tbuf0, lbuf0, ybuf0, ygbuf0),
             (b1, p1, tbuf1, lbuf1, ybuf1, ygbuf1))):

        def block_body(i, carry, g=g, b=b, p=p, tb=tb, lb=lb, yb=yb, ygb=ygb):
            glob = i + g * _NI
            par = lax.rem(glob, _RING)

            @pl.when(glob >= _RING)
            def _drain_one():
                pltpu.make_async_copy(local_hbm.at[pl.ds(0, 32)],
                                      outbuf.at[0], sem_val).wait()

            ifull = jnp.full((16,), i, jnp.int32)

            yr = [yb[i, pl.ds(0, 16)], yb[i, pl.ds(16, 16)]]
            ygr = [ygb[i, pl.ds(0, 16)], ygb[i, pl.ds(16, 16)]]

            def _row(j):
                y_s = yr[j // 16][j % 16]
                yg_s = ygr[j // 16][j % 16]
                yc = jnp.where(y_s == y_s, y_s, 0.0)
                gn = jnp.where(yg_s == yg_s, 0.0, 1.0)
                for d in range(_NCD):
                    t = tb[j, pl.ds(d * 16, 16)]
                    l = lb[i, pl.ds(d * 16, 16)]
                    outbuf[par, j, pl.ds(d * 16, 16)] = (
                        (t + l) + (yc * wy[d] + gn * gdiff[d]))

            for j in range(32):
                _row(j)
            n = p + 16 * i
            pltpu.async_copy(outbuf.at[par], val_out.at[b, pl.ds(n * 32, 32)],
                             sem_val)
            return carry

        lax.fori_loop(0, _NI, block_body, 0)

    for _ in range(_RING):
        pltpu.make_async_copy(local_hbm.at[pl.ds(0, 32)], outbuf.at[0],
                              sem_val).wait()
    for _ in range(_KT // _ROWS):
        pltpu.make_async_copy(local_hbm.at[pl.ds(0, _ROWS)],
                              rowbuf, sem_space).wait()
    pltpu.make_async_copy(var_out.at[0, pl.ds(0, _KT)], varbuf,
                          sem_space).wait()


def kernel(x, y, t2v_w, t2v_b, local_table, vt_w, vt_b, space_table,
           given_table):
    batch = x.shape[0]
    t2vw_f = t2v_w.reshape(1, _TD)
    t2vb_f = t2v_b.reshape(1, _TD)
    vtb_f = vt_b.reshape(1, _D)

    t_tab = pl.pallas_call(
        _tc_t_body,
        grid=(batch,),
        in_specs=[
            pl.BlockSpec((1, _N, _DX), lambda b: (b, 0, 0)),
            pl.BlockSpec((1, _TD), lambda b: (0, 0)),
            pl.BlockSpec((1, _TD), lambda b: (0, 0)),
            pl.BlockSpec((_TD + 1, _D), lambda b: (0, 0)),
            pl.BlockSpec((1, _D), lambda b: (0, 0)),
            pl.BlockSpec((2, _D), lambda b: (0, 0)),
        ],
        out_specs=pl.BlockSpec((1, _N, _D), lambda b: (b, 0, 0)),
        out_shape=jax.ShapeDtypeStruct((batch, _N, _D), jnp.float32),
    )(x, t2vw_f, t2vb_f, vt_w, vtb_f, given_table)

    y3 = y.reshape(batch, _N, _MAP * _DY)
    yperm = y3.reshape(batch, 32, 16, 32).transpose(0, 2, 1, 3)
    yg3 = jnp.transpose(y, (0, 1, 3, 2)).reshape(batch, _N, _MAP * _DY)
    ygperm = yg3.reshape(batch, 32, 16, 32).transpose(0, 2, 1, 3)
    wrow = vt_w[_TD:_TD + 1, :]

    sc_fill = functools.partial(
        pl.kernel,
        out_type=[
            jax.ShapeDtypeStruct((batch, _K, _D), jnp.float32),
            jax.ShapeDtypeStruct((batch, _K, _D), jnp.float32),
            jax.ShapeDtypeStruct((batch, _K), jnp.int32),
        ],
        mesh=plsc.VectorSubcoreMesh(core_axis_name="c", subcore_axis_name="s"),
        scratch_types=[
            pltpu.VMEM((32, _D), jnp.float32),
            pltpu.VMEM((32, _D), jnp.float32),
            pltpu.VMEM((32, _D), jnp.float32),
            pltpu.VMEM((32, _D), jnp.float32),
            pltpu.VMEM((32, 32), jnp.float32),
            pltpu.VMEM((32, 32), jnp.float32),
            pltpu.VMEM((32, 32), jnp.float32),
            pltpu.VMEM((32, 32), jnp.float32),
            pltpu.VMEM((_RING, 32, _D), jnp.float32),
            pltpu.VMEM((_ROWS, _D), jnp.float32),
            pltpu.VMEM((_KT,), jnp.int32),
            pltpu.VMEM((3, _D), jnp.float32),
            pltpu.VMEM((32,), jnp.int32),
            pltpu.VMEM((32,), jnp.int32),
            pltpu.SemaphoreType.DMA,
            pltpu.SemaphoreType.DMA,
            pltpu.SemaphoreType.DMA,
        ],
    )(_sc_body)
    val, space_emb, var_idx = sc_fill(t_tab, local_table, yperm, ygperm,
                                      space_table, wrow, given_table)
    return (val, space_emb, var_idx)


# speedup vs baseline: 1.8788x; 1.8788x over previous
import functools

import jax
import jax.numpy as jnp
from jax import lax
from jax.experimental import pallas as pl
from jax.experimental.pallas import tpu as pltpu
from jax.experimental.pallas import tpu_sc as plsc

_B, _N, _MAP, _DY, _DX = 4, 512, 4, 8, 6
_D = 256
_TE = 6
_TD = _TE * _DX
_K = _N * _MAP * _DY
_KT = 2048
_NBLK = _K // _KT
_NC, _NS = 2, 16
_ROWS = 64
_NCD = _D // 16
_NI = 32
_RING = 4


def _tc_t_body(x_ref, t2vw_ref, t2vb_ref, vtw_ref, vtb_ref, given_ref, t_ref):
    x = x_ref[0]
    xn = jnp.where(jnp.isnan(x), 0.0, x)
    xrep = jnp.repeat(xn, _TE, axis=1)
    xa = xrep * t2vw_ref[...] + t2vb_ref[...]
    col = lax.broadcasted_iota(jnp.int32, (_N, _TD), 1)
    tv = jnp.where(col % _TE == 0, xa, jnp.sin(xa))
    tt = jnp.dot(tv, vtw_ref[:_TD, :], preferred_element_type=jnp.float32)
    t_ref[0] = tt + vtb_ref[...] + given_ref[1:2, :]


def _sc_body(t_hbm, local_hbm, yperm_hbm, ygperm_hbm, space_hbm, wrow_hbm,
             given_hbm, val_out, space_out, var_out,
             tbuf0, tbuf1, lbuf0, lbuf1, ybuf0, ybuf1, ygbuf0, ygbuf1,
             outbuf, rowbuf, varbuf, gbuf, idxl0, idxl1,
             sem_pre, sem_val, sem_space):
    wid = lax.axis_index("s") * _NC + lax.axis_index("c")
    b0 = wid // 16
    p0 = lax.rem(wid, 16)
    b1 = (wid + 32) // 16
    p1 = lax.rem(wid + 32, 16)

    bs = wid // _NBLK
    seg = lax.rem(wid, _NBLK)
    pltpu.sync_copy(space_hbm.at[pl.ds(seg, 1)], rowbuf.at[pl.ds(0, 1)])
    svec = [rowbuf[0, pl.ds(d * 16, 16)] for d in range(_NCD)]
    for r in range(1, _ROWS):
        for d in range(_NCD):
            rowbuf[r, pl.ds(d * 16, 16)] = svec[d]
    vv = jnp.full((16,), seg, jnp.int32)
    for q in range(_KT // 16):
        varbuf[pl.ds(q * 16, 16)] = vv
    sbase = seg * _KT
    for i in range(_KT // _ROWS):
        pltpu.async_copy(rowbuf, space_out.at[bs, pl.ds(sbase + i * _ROWS,
                                                        _ROWS)], sem_space)
    pltpu.async_copy(varbuf, var_out.at[bs, pl.ds(sbase, _KT)], sem_space)

    cps = [pltpu.async_copy(wrow_hbm, gbuf.at[pl.ds(0, 1)], sem_pre),
           pltpu.async_copy(given_hbm, gbuf.at[pl.ds(1, 2)], sem_pre)]
    iv = jnp.arange(16, dtype=jnp.int32) * 16
    for g, (b, p, il, tb, lb, yb, ygb) in enumerate(
            ((b0, p0, idxl0, tbuf0, lbuf0, ybuf0, ygbuf0),
             (b1, p1, idxl1, tbuf1, lbuf1, ybuf1, ygbuf1))):
        il[pl.ds(0, 16)] = p + iv
        il[pl.ds(16, 16)] = p + 256 + iv
        cps.append(pltpu.async_copy(t_hbm.at[b, pl.ds(32 * p, 32)], tb,
                                    sem_pre))
        cps.append(pltpu.async_copy(local_hbm.at[il], lb, sem_pre))
        cps.append(pltpu.async_copy(yperm_hbm.at[b, p], yb, sem_pre))
        cps.append(pltpu.async_copy(ygperm_hbm.at[b, p], ygb, sem_pre))
    for cp in cps:
        cp.wait()
    wy = [gbuf[0, pl.ds(d * 16, 16)] for d in range(_NCD)]
    gdiff = [gbuf[1, pl.ds(d * 16, 16)] - gbuf[2, pl.ds(d * 16, 16)]
             for d in range(_NCD)]

    for g, (b, p, tb, lb, yb, ygb) in enumerate(
            ((b0, p0, tbuf0, lbuf0, ybuf0, ygbuf0),
             (b1, p1, tbuf1, lbuf1, ybuf1, ygbuf1))):

        def block_body(i, carry, g=g, b=b, p=p, tb=tb, lb=lb, yb=yb, ygb=ygb):
            glob = i + g * _NI
            par = lax.rem(glob, _RING)

            @pl.when(glob >= _RING)
            def _drain_one():
                pltpu.make_async_copy(local_hbm.at[pl.ds(0, 32)],
                                      outbuf.at[0], sem_val).wait()

            ifull = jnp.full((16,), i, jnp.int32)

            yr = [yb[i, pl.ds(0, 16)], yb[i, pl.ds(16, 16)]]
            ygr = [ygb[i, pl.ds(0, 16)], ygb[i, pl.ds(16, 16)]]
            for j in range(32):
                y_s = yr[j // 16][j % 16]
                yg_s = ygr[j // 16][j % 16]
                yc = jnp.where(y_s == y_s, y_s, 0.0)
                gn = jnp.where(yg_s == yg_s, 0.0, 1.0)
                for h in range(2):
                    d0 = h * (_NCD // 2)
                    ts = [tb[j, pl.ds((d0 + d) * 16, 16)]
                          for d in range(_NCD // 2)]
                    accs = [(ts[d] + lb[i, pl.ds((d0 + d) * 16, 16)])
                            + (yc * wy[d0 + d] + gn * gdiff[d0 + d])
                            for d in range(_NCD // 2)]
                    for d in range(_NCD // 2):
                        outbuf[par, j, pl.ds((d0 + d) * 16, 16)] = accs[d]

            n = p + 16 * i
            pltpu.async_copy(outbuf.at[par], val_out.at[b, pl.ds(n * 32, 32)],
                             sem_val)
            return carry

        lax.fori_loop(0, _NI, block_body, 0)

    for _ in range(_RING):
        pltpu.make_async_copy(local_hbm.at[pl.ds(0, 32)], outbuf.at[0],
                              sem_val).wait()
    for _ in range(_KT // _ROWS):
        pltpu.make_async_copy(local_hbm.at[pl.ds(0, _ROWS)],
                              rowbuf, sem_space).wait()
    pltpu.make_async_copy(var_out.at[0, pl.ds(0, _KT)], varbuf,
                          sem_space).wait()


def kernel(x, y, t2v_w, t2v_b, local_table, vt_w, vt_b, space_table,
           given_table):
    batch = x.shape[0]
    t2vw_f = t2v_w.reshape(1, _TD)
    t2vb_f = t2v_b.reshape(1, _TD)
    vtb_f = vt_b.reshape(1, _D)

    t_tab = pl.pallas_call(
        _tc_t_body,
        grid=(batch,),
        in_specs=[
            pl.BlockSpec((1, _N, _DX), lambda b: (b, 0, 0)),
            pl.BlockSpec((1, _TD), lambda b: (0, 0)),
            pl.BlockSpec((1, _TD), lambda b: (0, 0)),
            pl.BlockSpec((_TD + 1, _D), lambda b: (0, 0)),
            pl.BlockSpec((1, _D), lambda b: (0, 0)),
            pl.BlockSpec((2, _D), lambda b: (0, 0)),
        ],
        out_specs=pl.BlockSpec((1, _N, _D), lambda b: (b, 0, 0)),
        out_shape=jax.ShapeDtypeStruct((batch, _N, _D), jnp.float32),
    )(x, t2vw_f, t2vb_f, vt_w, vtb_f, given_table)

    y3 = y.reshape(batch, _N, _MAP * _DY)
    yperm = y3.reshape(batch, 32, 16, 32).transpose(0, 2, 1, 3)
    yg3 = jnp.transpose(y, (0, 1, 3, 2)).reshape(batch, _N, _MAP * _DY)
    ygperm = yg3.reshape(batch, 32, 16, 32).transpose(0, 2, 1, 3)
    wrow = vt_w[_TD:_TD + 1, :]

    sc_fill = functools.partial(
        pl.kernel,
        out_type=[
            jax.ShapeDtypeStruct((batch, _K, _D), jnp.float32),
            jax.ShapeDtypeStruct((batch, _K, _D), jnp.float32),
            jax.ShapeDtypeStruct((batch, _K), jnp.int32),
        ],
        mesh=plsc.VectorSubcoreMesh(core_axis_name="c", subcore_axis_name="s"),
        scratch_types=[
            pltpu.VMEM((32, _D), jnp.float32),
            pltpu.VMEM((32, _D), jnp.float32),
            pltpu.VMEM((32, _D), jnp.float32),
            pltpu.VMEM((32, _D), jnp.float32),
            pltpu.VMEM((32, 32), jnp.float32),
            pltpu.VMEM((32, 32), jnp.float32),
            pltpu.VMEM((32, 32), jnp.float32),
            pltpu.VMEM((32, 32), jnp.float32),
            pltpu.VMEM((_RING, 32, _D), jnp.float32),
            pltpu.VMEM((_ROWS, _D), jnp.float32),
            pltpu.VMEM((_KT,), jnp.int32),
            pltpu.VMEM((3, _D), jnp.float32),
            pltpu.VMEM((32,), jnp.int32),
            pltpu.VMEM((32,), jnp.int32),
            pltpu.SemaphoreType.DMA,
            pltpu.SemaphoreType.DMA,
            pltpu.SemaphoreType.DMA,
        ],
    )(_sc_body)
    val, space_emb, var_idx = sc_fill(t_tab, local_table, yperm, ygperm,
                                      space_table, wrow, given_table)
    return (val, space_emb, var_idx)


# speedup vs baseline: 3.1480x; 1.6755x over previous
import jax
import jax.numpy as jnp
from jax import lax
from jax.experimental import pallas as pl

_B, _N, _MAP, _DY, _DX = 4, 512, 4, 8, 6
_D = 256
_TE = 6
_TD = _TE * _DX
_K = _N * _MAP * _DY
_KT = 2048
_NBLK = _K // _KT


def _tc_t_body(x_ref, t2vw_ref, t2vb_ref, vtw_ref, vtb_ref, given_ref, t_ref):
    x = x_ref[0]
    xn = jnp.where(jnp.isnan(x), 0.0, x)
    xrep = jnp.repeat(xn, _TE, axis=1)
    xa = xrep * t2vw_ref[...] + t2vb_ref[...]
    col = lax.broadcasted_iota(jnp.int32, (_N, _TD), 1)
    tv = jnp.where(col % _TE == 0, xa, jnp.sin(xa))
    tt = jnp.dot(tv, vtw_ref[:_TD, :], preferred_element_type=jnp.float32)
    t_ref[0] = tt + vtb_ref[...] + given_ref[1:2, :]


def _asm_body(t_ref, y_ref, yg_ref, local_ref, wrow_ref, gdif_ref, space_ref,
              val_ref, space_out_ref, var_ref):
    c = pl.program_id(1)
    t_exp = jnp.tile(t_ref[0], (_KT // _N, 1))
    local_exp = jnp.repeat(local_ref[...], 32, axis=0)
    yv = y_ref[0, 0]
    yc = jnp.where(jnp.isnan(yv), 0.0, yv)
    gmask = jnp.isnan(yg_ref[0, 0])
    gcor = jnp.where(gmask, gdif_ref[...], 0.0)
    val_ref[0] = t_exp + local_exp + yc * wrow_ref[...] + gcor
    rows = space_ref[...]
    rsel = lax.broadcasted_iota(jnp.int32, (_DY, 1), 0) == c
    srow = jnp.sum(jnp.where(rsel, rows, 0.0), axis=0, keepdims=True)
    space_out_ref[0] = jnp.broadcast_to(srow, (_KT, _D))
    var_ref[0, 0] = jnp.full((1, _KT), c, jnp.int32)


def kernel(x, y, t2v_w, t2v_b, local_table, vt_w, vt_b, space_table,
           given_table):
    batch = x.shape[0]
    t2vw_f = t2v_w.reshape(1, _TD)
    t2vb_f = t2v_b.reshape(1, _TD)
    vtb_f = vt_b.reshape(1, _D)

    t_tab = pl.pallas_call(
        _tc_t_body,
        grid=(batch,),
        in_specs=[
            pl.BlockSpec((1, _N, _DX), lambda b: (b, 0, 0)),
            pl.BlockSpec((1, _TD), lambda b: (0, 0)),
            pl.BlockSpec((1, _TD), lambda b: (0, 0)),
            pl.BlockSpec((_TD + 1, _D), lambda b: (0, 0)),
            pl.BlockSpec((1, _D), lambda b: (0, 0)),
            pl.BlockSpec((2, _D), lambda b: (0, 0)),
        ],
        out_specs=pl.BlockSpec((1, _N, _D), lambda b: (b, 0, 0)),
        out_shape=jax.ShapeDtypeStruct((batch, _N, _D), jnp.float32),
    )(x, t2vw_f, t2vb_f, vt_w, vtb_f, given_table)

    y_flat = y.reshape(batch, _NBLK, _KT, 1)
    yg_flat = jnp.transpose(y, (0, 1, 3, 2)).reshape(batch, _NBLK, _KT, 1)
    wrow = vt_w[_TD:_TD + 1, :]
    gdif = (given_table[0:1, :] - given_table[1:2, :])

    val, space_emb, var4 = pl.pallas_call(
        _asm_body,
        grid=(batch, _NBLK),
        in_specs=[
            pl.BlockSpec((1, _N, _D), lambda b, c: (b, 0, 0)),
            pl.BlockSpec((1, 1, _KT, 1), lambda b, c: (b, c, 0, 0)),
            pl.BlockSpec((1, 1, _KT, 1), lambda b, c: (b, c, 0, 0)),
            pl.BlockSpec((_KT // 32, _D), lambda b, c: (c, 0)),
            pl.BlockSpec((1, _D), lambda b, c: (0, 0)),
            pl.BlockSpec((1, _D), lambda b, c: (0, 0)),
            pl.BlockSpec((_DY, _D), lambda b, c: (0, 0)),
        ],
        out_specs=[
            pl.BlockSpec((1, _KT, _D), lambda b, c: (b, c, 0)),
            pl.BlockSpec((1, _KT, _D), lambda b, c: (b, c, 0)),
            pl.BlockSpec((1, 1, 1, _KT), lambda b, c: (b, c, 0, 0)),
        ],
        out_shape=[
            jax.ShapeDtypeStruct((batch, _K, _D), jnp.float32),
            jax.ShapeDtypeStruct((batch, _K, _D), jnp.float32),
            jax.ShapeDtypeStruct((batch, _NBLK, 1, _KT), jnp.int32),
        ],
    )(t_tab, y_flat, yg_flat, local_table, wrow, gdif, space_table)
    return (val, space_emb, var4.reshape(batch, _K))
